# baseline (speedup 1.0000x reference)
import functools

import jax
import jax.numpy as jnp
from jax import lax
from jax.experimental import pallas as pl
from jax.experimental.pallas import tpu as pltpu
from jax.experimental.pallas import tpu_sc as plsc

N = 10000
IN_DIM = 128
LAT = 256
E = 320000

NC = 2
NS = 16
NW = NC * NS
LANES = 16

EPW = E // NW
CHUNK_A = 80
NCHUNK_A = EPW // CHUNK_A

ROWS_PER = (N + NW - 1) // NW
CAP = 128
OCAP = 512
PKSH = 16384


def _lrelu(a):
    return jnp.where(a >= 0, a, 0.01 * a)


def _sload(ref, idx):
    return ref[pl.ds(idx, LANES)][0]


def _encode_body(x_ref, w1_ref, b1_ref, w2_ref, h_ref, g_ref):
    h = _lrelu(jnp.dot(x_ref[...], w1_ref[...],
                       preferred_element_type=jnp.float32) + b1_ref[...])
    h_ref[...] = h
    g_ref[...] = jnp.dot(h, w2_ref[...], preferred_element_type=jnp.float32)


def _encode(x, W1, b1, W2):
    RB = 1000
    grid = (N // RB,)
    return pl.pallas_call(
        _encode_body,
        grid=grid,
        in_specs=[
            pl.BlockSpec((RB, IN_DIM), lambda i: (i, 0)),
            pl.BlockSpec((IN_DIM, LAT), lambda i: (0, 0)),
            pl.BlockSpec((1, LAT), lambda i: (0, 0)),
            pl.BlockSpec((LAT, LAT), lambda i: (0, 0)),
        ],
        out_specs=[
            pl.BlockSpec((RB, LAT), lambda i: (i, 0)),
            pl.BlockSpec((RB, LAT), lambda i: (i, 0)),
        ],
        out_shape=[
            jax.ShapeDtypeStruct((N, LAT), jnp.float32),
            jax.ShapeDtypeStruct((N, LAT), jnp.float32),
        ],
    )(x, W1, b1.reshape(1, LAT), W2)


def _scores_body(g_hbm, src_hbm, dst_hbm, b2_hbm,
                 bpack_hbm, bval_hbm, bcnt_hbm,
                 idx_u0, idx_v0, urows0, vrows0,
                 idx_u1, idx_v1, urows1, vrows1,
                 b2_v, pkbuf, vlbuf, bcnt, sem0, sem1):
    c = lax.axis_index("c")
    s = lax.axis_index("s")
    wid = s * NC + c
    base = wid * EPW
    pltpu.sync_copy(b2_hbm, b2_v)
    iota = lax.iota(jnp.int32, LANES)

    occ_cal, _ = plsc.scan_count(jnp.zeros((LANES,), jnp.int32))
    occ_base = jnp.min(occ_cal)
    for i in range(NW // LANES):
        bcnt[pl.ds(i * LANES, LANES)] = jnp.zeros((LANES,), jnp.int32)

    bufs = ((idx_u0, idx_v0, urows0, vrows0, sem0),
            (idx_u1, idx_v1, urows1, vrows1, sem1))

    def issue(ci, b):
        iu, iv, ur, vr, sm = bufs[b]
        eb = base + ci * CHUNK_A
        pltpu.sync_copy(src_hbm.at[pl.ds(eb, CHUNK_A)], iu)
        pltpu.sync_copy(dst_hbm.at[pl.ds(eb, CHUNK_A)], iv)
        pltpu.async_copy(g_hbm.at[iu], ur, sm)
        pltpu.async_copy(g_hbm.at[iv], vr, sm)

    def wait(b):
        iu, iv, ur, vr, sm = bufs[b]
        pltpu.make_async_copy(g_hbm.at[iu], ur, sm).wait()
        pltpu.make_async_copy(g_hbm.at[iv], vr, sm).wait()

    def compute(ci, b):
        iu, iv, ur, vr, _ = bufs[b]
        for gg in range(CHUNK_A // LANES):
            def pair_body(l, sumv):
                e0 = gg * LANES + 2 * l
                e1 = e0 + 1
                acc0 = jnp.zeros((LANES,), jnp.float32)
                acc1 = jnp.zeros((LANES,), jnp.float32)
                for j in range(LAT // LANES):
                    sl = pl.ds(j * LANES, LANES)
                    bj = b2_v[sl]
                    t0 = ur[e0, sl] - vr[e0, sl] + bj
                    t1 = ur[e1, sl] - vr[e1, sl] + bj
                    acc0 = acc0 + jnp.where(t0 >= 0, t0, 0.01 * t0)
                    acc1 = acc1 + jnp.where(t1 >= 0, t1, 0.01 * t1)
                s0 = jnp.sum(acc0)
                s1 = jnp.sum(acc1)
                return (sumv + jnp.where(iota == 2 * l, s0, 0.0)
                             + jnp.where(iota == 2 * l + 1, s1, 0.0))
            sumv = lax.fori_loop(0, LANES // 2, pair_body,
                                 jnp.zeros((LANES,), jnp.float32))
            sig = 1.0 / (1.0 + jnp.exp(-sumv))
            s16 = iu[pl.ds(gg * LANES, LANES)]
            d16 = iv[pl.ds(gg * LANES, LANES)]
            owner = lax.div(s16, ROWS_PER)
            loc = s16 - owner * ROWS_PER
            pack = loc * PKSH + d16
            occ, lastm = plsc.scan_count(owner)
            occ0 = occ - occ_base
            old = plsc.load_gather(bcnt, [owner])
            pos = old + occ0
            ok = pos < OCAP
            bidx = jnp.where(ok, owner * OCAP + pos, 0)
            plsc.store_scatter(pkbuf, [bidx], pack, mask=ok)
            plsc.store_scatter(vlbuf, [bidx], sig, mask=ok)
            plsc.addupdate_scatter(bcnt, [owner], occ0 + 1, mask=lastm)

    issue(0, 0)

    def pair_chunks(p, _):
        c0 = 2 * p
        c1 = 2 * p + 1
        wait(0)

        @pl.when(c1 < NCHUNK_A)
        def _():
            issue(c1, 1)
        compute(c0, 0)

        @pl.when(c1 < NCHUNK_A)
        def _():
            wait(1)

            @pl.when(c1 + 1 < NCHUNK_A)
            def _():
                issue(c1 + 1, 0)
            compute(c1, 1)
        return 0

    lax.fori_loop(0, (NCHUNK_A + 1) // 2, pair_chunks, 0)

    pltpu.sync_copy(pkbuf, bpack_hbm.at[pl.ds(wid * NW * OCAP, NW * OCAP)])
    pltpu.sync_copy(vlbuf, bval_hbm.at[pl.ds(wid * NW * OCAP, NW * OCAP)])
    pltpu.sync_copy(bcnt.at[pl.ds(0, NW)], bcnt_hbm.at[pl.ds(wid * NW, NW)])


def _scores(g, e_src, e_dst, b2):
    mesh = plsc.VectorSubcoreMesh(core_axis_name="c", subcore_axis_name="s",
                                  num_cores=NC, num_subcores=NS)
    f = pl.kernel(
        _scores_body,
        out_type=[
            jax.ShapeDtypeStruct((NW * NW * OCAP,), jnp.int32),
            jax.ShapeDtypeStruct((NW * NW * OCAP,), jnp.float32),
            jax.ShapeDtypeStruct((NW * NW,), jnp.int32),
        ],
        mesh=mesh,
        compiler_params=pltpu.CompilerParams(needs_layout_passes=False),
        scratch_types=[
            pltpu.VMEM((CHUNK_A,), jnp.int32),
            pltpu.VMEM((CHUNK_A,), jnp.int32),
            pltpu.VMEM((CHUNK_A, LAT), jnp.float32),
            pltpu.VMEM((CHUNK_A, LAT), jnp.float32),
            pltpu.VMEM((CHUNK_A,), jnp.int32),
            pltpu.VMEM((CHUNK_A,), jnp.int32),
            pltpu.VMEM((CHUNK_A, LAT), jnp.float32),
            pltpu.VMEM((CHUNK_A, LAT), jnp.float32),
            pltpu.VMEM((LAT,), jnp.float32),
            pltpu.VMEM((NW * OCAP,), jnp.int32),
            pltpu.VMEM((NW * OCAP,), jnp.float32),
            pltpu.VMEM((NW + LANES,), jnp.int32),
            pltpu.SemaphoreType.DMA,
            pltpu.SemaphoreType.DMA,
        ],
    )
    return f(g, e_src, e_dst, b2)


def _rank_body(bpack_hbm, bval_hbm, bcnt_hbm, params_hbm, out_hbm,
               pk_c, vl_c, bcv, gdst, gval, cnt, svm,
               rowbuf0, rowbuf1, params_v, semr0, semr1):
    c = lax.axis_index("c")
    s = lax.axis_index("s")
    wid = s * NC + c
    lo = wid * ROWS_PER
    nrows = jnp.minimum(ROWS_PER, N - lo)
    iota = lax.iota(jnp.int32, LANES)
    zeros16 = jnp.zeros((LANES,), jnp.float32)

    pltpu.sync_copy(params_hbm, params_v)
    pv = params_v[...]
    w3s = pv[0]
    b3s = pv[1]

    occ_cal, _ = plsc.scan_count(jnp.zeros((LANES,), jnp.int32))
    occ_base = jnp.min(occ_cal)

    for i in range(320 // LANES):
        cnt[pl.ds(i * LANES, LANES)] = jnp.zeros((LANES,), jnp.int32)
    def zrow(i, _):
        rowbuf0[pl.ds(i * LANES, LANES)] = zeros16
        rowbuf1[pl.ds(i * LANES, LANES)] = zeros16
        return 0
    lax.fori_loop(0, N // LANES, zrow, 0)

    pltpu.sync_copy(bcnt_hbm, bcv.at[pl.ds(0, NW * NW)])

    def prod_body(p, _):
        cp = jnp.minimum(_sload(bcv, p * NW + wid), OCAP)
        seg = p * NW * OCAP + wid * OCAP
        pltpu.sync_copy(bpack_hbm.at[pl.ds(seg, OCAP)], pk_c)
        pltpu.sync_copy(bval_hbm.at[pl.ds(seg, OCAP)], vl_c)
        ng = (cp + LANES - 1) // LANES

        def grp_body(gg, _):
            off = gg * LANES
            valid = (off + iota) < cp
            pk16 = pk_c[pl.ds(off, LANES)]
            v16 = vl_c[pl.ds(off, LANES)]
            locv = jnp.where(valid, lax.shift_right_arithmetic(pk16, 14), 0)
            d16 = pk16 & (PKSH - 1)
            occ, lastm = plsc.scan_count(locv, mask=valid)
            occ0 = occ - occ_base
            old = plsc.load_gather(cnt, [locv], mask=valid)
            pos = old + occ0
            ok = valid & (pos < CAP)
            gidx = jnp.where(ok, locv * CAP + pos, 0)
            plsc.store_scatter(gdst, [gidx], d16, mask=ok)
            plsc.store_scatter(gval, [gidx], v16, mask=ok)
            plsc.addupdate_scatter(cnt, [locv], occ0 + 1, mask=lastm & valid)
            return 0

        lax.fori_loop(0, ng, grp_body, 0)
        return 0

    lax.fori_loop(0, NW, prod_body, 0)

    rbufs = ((rowbuf0, semr0), (rowbuf1, semr1))

    def process_row(r, rowbuf):
        m = jnp.minimum(_sload(cnt, r), CAP)
        o = r * CAP
        nblk = (m + LANES - 1) // LANES

        def rsum(jb, acc):
            vv = (jb * LANES + iota) < m
            return acc + jnp.where(vv, gval[pl.ds(o + jb * LANES, LANES)], 0.0)
        rs = jnp.sum(lax.fori_loop(0, nblk, rsum, zeros16))
        kk = w3s * rs + b3s
        kk = jnp.where(kk >= 0, kk, 0.01 * kk)

        def b1_blk(jb, _):
            jidx = jb * LANES + iota
            jv = jidx < m
            dstj = gdst[pl.ds(o + jb * LANES, LANES)]

            def b1(j2, carry):
                sv16, first16 = carry
                dp = _sload(gdst, o + j2)
                vp = _sload(gval, o + j2)
                eq = jv & (dstj == dp)
                sv16 = sv16 + jnp.where(eq, vp, 0.0)
                first16 = jnp.minimum(first16,
                                      jnp.where(eq, j2, jnp.int32(1 << 30)))
                return sv16, first16

            sv16, first16 = lax.fori_loop(
                0, m, b1,
                (zeros16, jnp.full((LANES,), 1 << 30, jnp.int32)))
            keep16 = jv & (first16 == jidx)
            svm[pl.ds(jb * LANES, LANES)] = jnp.where(keep16, sv16, -1.0)
            return 0

        lax.fori_loop(0, nblk, b1_blk, 0)

        def b2_blk(jb, _):
            dstj = gdst[pl.ds(o + jb * LANES, LANES)]
            svj = svm[pl.ds(jb * LANES, LANES)]
            alive = svj > 0

            def b2(j2, rank16):
                sp = _sload(svm, j2)
                dp = _sload(gdst, o + j2)
                gt = sp > svj
                tie = (sp == svj) & (dp < dstj)
                return rank16 + jnp.where(gt | tie, 1.0, 0.0)

            rank16 = lax.fori_loop(0, m, b2, zeros16)
            t16 = rank16 - kk
            e2 = jnp.exp(2.0 * t16)
            th = 1.0 - 2.0 / (e2 + 1.0)
            out16 = svj * (1.5 - 0.5 * th)
            plsc.store_scatter(rowbuf, [dstj], out16, mask=alive)
            return 0

        lax.fori_loop(0, nblk, b2_blk, 0)

    def zero_stale(r, rowbuf):
        m = jnp.minimum(_sload(cnt, r), CAP)
        o = r * CAP
        nblk = (m + LANES - 1) // LANES

        def zb(jb, _):
            jv = (jb * LANES + iota) < m
            dstj = gdst[pl.ds(o + jb * LANES, LANES)]
            plsc.store_scatter(rowbuf, [dstj], zeros16, mask=jv)
            return 0

        lax.fori_loop(0, nblk, zb, 0)

    def wait_row(b):
        rowbuf, sem = rbufs[b]
        pltpu.make_async_copy(rowbuf, out_hbm.at[lo], sem).wait()

    def handle_row(p, r, b):
        rowbuf, sem = rbufs[b]

        @pl.when(p > 0)
        def _():
            wait_row(b)
            zero_stale(r - 2, rowbuf)

        process_row(r, rowbuf)
        pltpu.async_copy(rowbuf, out_hbm.at[lo + r], sem)

    def pair_body(p, _):
        r0 = 2 * p
        handle_row(p, r0, 0)

        @pl.when(r0 + 1 < nrows)
        def _():
            handle_row(p, r0 + 1, 1)
        return 0

    lax.fori_loop(0, (nrows + 1) // 2, pair_body, 0)
    wait_row(0)

    @pl.when(nrows > 1)
    def _():
        wait_row(1)


def _rank_scatter(bpack, bval, bcnt, params):
    mesh = plsc.VectorSubcoreMesh(core_axis_name="c", subcore_axis_name="s",
                                  num_cores=NC, num_subcores=NS)
    f = pl.kernel(
        _rank_body,
        out_type=jax.ShapeDtypeStruct((N, N), jnp.float32),
        mesh=mesh,
        compiler_params=pltpu.CompilerParams(needs_layout_passes=False),
        scratch_types=[
            pltpu.VMEM((OCAP,), jnp.int32),
            pltpu.VMEM((OCAP,), jnp.float32),
            pltpu.VMEM((NW * NW + LANES,), jnp.int32),
            pltpu.VMEM((ROWS_PER * CAP + LANES,), jnp.int32),
            pltpu.VMEM((ROWS_PER * CAP + LANES,), jnp.float32),
            pltpu.VMEM((336,), jnp.int32),
            pltpu.VMEM((CAP + LANES,), jnp.float32),
            pltpu.VMEM((N,), jnp.float32),
            pltpu.VMEM((N,), jnp.float32),
            pltpu.VMEM((LANES,), jnp.float32),
            pltpu.SemaphoreType.DMA,
            pltpu.SemaphoreType.DMA,
        ],
    )
    return f(bpack, bval, bcnt, params)


def kernel(x, edge_index, W1, b1, W2, b2, W3, b3):
    h, g = _encode(x, W1, b1, W2)
    e_src = edge_index[0]
    e_dst = edge_index[1]
    bpack, bval, bcnt = _scores(g, e_src, e_dst, b2)
    params = jnp.zeros((LANES,), jnp.float32)
    params = params.at[0].set(W3[0, 0]).at[1].set(b3[0])
    out = _rank_scatter(bpack, bval, bcnt, params)
    return out, h


# speedup vs baseline: 1.1862x; 1.1862x over previous
import functools

import jax
import jax.numpy as jnp
from jax import lax
from jax.experimental import pallas as pl
from jax.experimental.pallas import tpu as pltpu
from jax.experimental.pallas import tpu_sc as plsc

N = 10000
IN_DIM = 128
LAT = 256
E = 320000

NC = 2
NS = 16
NW = NC * NS
LANES = 16

EPW = E // NW
CHUNK_A = 80
NCHUNK_A = EPW // CHUNK_A

ROWS_PER = (N + NW - 1) // NW
CAP = 128
OCAP = 512
PKSH = 16384


def _lrelu(a):
    return jnp.where(a >= 0, a, 0.01 * a)


def _sload(ref, idx):
    return ref[pl.ds(idx, LANES)][0]


def _encode_body(x_ref, w1_ref, b1_ref, w2_ref, h_ref, g_ref):
    h = _lrelu(jnp.dot(x_ref[...], w1_ref[...],
                       preferred_element_type=jnp.float32) + b1_ref[...])
    h_ref[...] = h
    g_ref[...] = jnp.dot(h, w2_ref[...], preferred_element_type=jnp.float32)


def _encode(x, W1, b1, W2):
    RB = 1000
    grid = (N // RB,)
    return pl.pallas_call(
        _encode_body,
        grid=grid,
        in_specs=[
            pl.BlockSpec((RB, IN_DIM), lambda i: (i, 0)),
            pl.BlockSpec((IN_DIM, LAT), lambda i: (0, 0)),
            pl.BlockSpec((1, LAT), lambda i: (0, 0)),
            pl.BlockSpec((LAT, LAT), lambda i: (0, 0)),
        ],
        out_specs=[
            pl.BlockSpec((RB, LAT), lambda i: (i, 0)),
            pl.BlockSpec((RB, LAT), lambda i: (i, 0)),
        ],
        out_shape=[
            jax.ShapeDtypeStruct((N, LAT), jnp.float32),
            jax.ShapeDtypeStruct((N, LAT), jnp.float32),
        ],
    )(x, W1, b1.reshape(1, LAT), W2)


def _scores_body(g_hbm, epk_hbm, b2_hbm,
                 bpack_hbm, bval_hbm, bcnt_hbm,
                 pkc0, idx_u0, idx_v0, urows0, vrows0,
                 pkc1, idx_u1, idx_v1, urows1, vrows1,
                 b2_v, pkbuf, vlbuf, bcnt, sem0, sem1):
    c = lax.axis_index("c")
    s = lax.axis_index("s")
    wid = s * NC + c
    base = wid * EPW
    pltpu.sync_copy(b2_hbm, b2_v)
    iota = lax.iota(jnp.int32, LANES)

    occ_cal, _ = plsc.scan_count(jnp.zeros((LANES,), jnp.int32))
    occ_base = jnp.min(occ_cal)
    for i in range(NW // LANES):
        bcnt[pl.ds(i * LANES, LANES)] = jnp.zeros((LANES,), jnp.int32)

    bufs = ((pkc0, idx_u0, idx_v0, urows0, vrows0, sem0),
            (pkc1, idx_u1, idx_v1, urows1, vrows1, sem1))

    def issue(ci, b):
        pkc, iu, iv, ur, vr, sm = bufs[b]
        eb = base + ci * CHUNK_A
        pltpu.sync_copy(epk_hbm.at[pl.ds(eb, CHUNK_A)], pkc)
        for gg in range(CHUNK_A // LANES):
            sl = pl.ds(gg * LANES, LANES)
            pk16 = pkc[sl]
            iu[sl] = lax.shift_right_arithmetic(pk16, 14)
            iv[sl] = pk16 & (PKSH - 1)
        pltpu.async_copy(g_hbm.at[iu], ur, sm)
        pltpu.async_copy(g_hbm.at[iv], vr, sm)

    def wait(b):
        _, iu, iv, ur, vr, sm = bufs[b]
        pltpu.make_async_copy(g_hbm.at[iu], ur, sm).wait()
        pltpu.make_async_copy(g_hbm.at[iv], vr, sm).wait()

    def compute(ci, b):
        _, iu, iv, ur, vr, _ = bufs[b]
        for gg in range(CHUNK_A // LANES):
            def pair_body(l, sumv):
                e0 = gg * LANES + 2 * l
                e1 = e0 + 1
                acc0 = jnp.zeros((LANES,), jnp.float32)
                acc1 = jnp.zeros((LANES,), jnp.float32)
                for j in range(LAT // LANES):
                    sl = pl.ds(j * LANES, LANES)
                    bj = b2_v[sl]
                    t0 = ur[e0, sl] - vr[e0, sl] + bj
                    t1 = ur[e1, sl] - vr[e1, sl] + bj
                    acc0 = acc0 + jnp.where(t0 >= 0, t0, 0.01 * t0)
                    acc1 = acc1 + jnp.where(t1 >= 0, t1, 0.01 * t1)
                s0 = jnp.sum(acc0)
                s1 = jnp.sum(acc1)
                return (sumv + jnp.where(iota == 2 * l, s0, 0.0)
                             + jnp.where(iota == 2 * l + 1, s1, 0.0))
            sumv = lax.fori_loop(0, LANES // 2, pair_body,
                                 jnp.zeros((LANES,), jnp.float32))
            sig = 1.0 / (1.0 + jnp.exp(-sumv))
            s16 = iu[pl.ds(gg * LANES, LANES)]
            d16 = iv[pl.ds(gg * LANES, LANES)]
            owner = lax.div(s16, ROWS_PER)
            loc = s16 - owner * ROWS_PER
            pack = loc * PKSH + d16
            occ, lastm = plsc.scan_count(owner)
            occ0 = occ - occ_base
            old = plsc.load_gather(bcnt, [owner])
            pos = old + occ0
            ok = pos < OCAP
            bidx = jnp.where(ok, owner * OCAP + pos, 0)
            plsc.store_scatter(pkbuf, [bidx], pack, mask=ok)
            plsc.store_scatter(vlbuf, [bidx], sig, mask=ok)
            plsc.addupdate_scatter(bcnt, [owner], occ0 + 1, mask=lastm)

    issue(0, 0)

    def pair_chunks(p, _):
        c0 = 2 * p
        c1 = 2 * p + 1
        wait(0)

        @pl.when(c1 < NCHUNK_A)
        def _():
            issue(c1, 1)
        compute(c0, 0)

        @pl.when(c1 < NCHUNK_A)
        def _():
            wait(1)

            @pl.when(c1 + 1 < NCHUNK_A)
            def _():
                issue(c1 + 1, 0)
            compute(c1, 1)
        return 0

    lax.fori_loop(0, (NCHUNK_A + 1) // 2, pair_chunks, 0)

    pltpu.sync_copy(pkbuf, bpack_hbm.at[pl.ds(wid * NW * OCAP, NW * OCAP)])
    pltpu.sync_copy(vlbuf, bval_hbm.at[pl.ds(wid * NW * OCAP, NW * OCAP)])
    pltpu.sync_copy(bcnt.at[pl.ds(0, NW)], bcnt_hbm.at[pl.ds(wid * NW, NW)])


def _scores(g, epk, b2):
    mesh = plsc.VectorSubcoreMesh(core_axis_name="c", subcore_axis_name="s",
                                  num_cores=NC, num_subcores=NS)
    f = pl.kernel(
        _scores_body,
        out_type=[
            jax.ShapeDtypeStruct((NW * NW * OCAP,), jnp.int32),
            jax.ShapeDtypeStruct((NW * NW * OCAP,), jnp.float32),
            jax.ShapeDtypeStruct((NW * NW,), jnp.int32),
        ],
        mesh=mesh,
        compiler_params=pltpu.CompilerParams(needs_layout_passes=False),
        scratch_types=[
            pltpu.VMEM((CHUNK_A,), jnp.int32),
            pltpu.VMEM((CHUNK_A,), jnp.int32),
            pltpu.VMEM((CHUNK_A,), jnp.int32),
            pltpu.VMEM((CHUNK_A, LAT), jnp.float32),
            pltpu.VMEM((CHUNK_A, LAT), jnp.float32),
            pltpu.VMEM((CHUNK_A,), jnp.int32),
            pltpu.VMEM((CHUNK_A,), jnp.int32),
            pltpu.VMEM((CHUNK_A,), jnp.int32),
            pltpu.VMEM((CHUNK_A, LAT), jnp.float32),
            pltpu.VMEM((CHUNK_A, LAT), jnp.float32),
            pltpu.VMEM((LAT,), jnp.float32),
            pltpu.VMEM((NW * OCAP,), jnp.int32),
            pltpu.VMEM((NW * OCAP,), jnp.float32),
            pltpu.VMEM((NW + LANES,), jnp.int32),
            pltpu.SemaphoreType.DMA,
            pltpu.SemaphoreType.DMA,
        ],
    )
    return f(g, epk, b2)


def _rank_body(bpack_hbm, bval_hbm, bcnt_hbm, params_hbm, out_hbm,
               pk_c, vl_c, bcv, gdst, gval, cnt, svm,
               rowbuf0, rowbuf1, params_v, semr0, semr1):
    c = lax.axis_index("c")
    s = lax.axis_index("s")
    wid = s * NC + c
    lo = wid * ROWS_PER
    nrows = jnp.minimum(ROWS_PER, N - lo)
    iota = lax.iota(jnp.int32, LANES)
    zeros16 = jnp.zeros((LANES,), jnp.float32)

    pltpu.sync_copy(params_hbm, params_v)
    pv = params_v[...]
    w3s = pv[0]
    b3s = pv[1]

    occ_cal, _ = plsc.scan_count(jnp.zeros((LANES,), jnp.int32))
    occ_base = jnp.min(occ_cal)

    for i in range(320 // LANES):
        cnt[pl.ds(i * LANES, LANES)] = jnp.zeros((LANES,), jnp.int32)
    def zrow(i, _):
        rowbuf0[pl.ds(i * LANES, LANES)] = zeros16
        rowbuf1[pl.ds(i * LANES, LANES)] = zeros16
        return 0
    lax.fori_loop(0, N // LANES, zrow, 0)

    pltpu.sync_copy(bcnt_hbm, bcv.at[pl.ds(0, NW * NW)])

    def prod_body(p, _):
        cp = jnp.minimum(_sload(bcv, p * NW + wid), OCAP)
        seg = p * NW * OCAP + wid * OCAP
        pltpu.sync_copy(bpack_hbm.at[pl.ds(seg, OCAP)], pk_c)
        pltpu.sync_copy(bval_hbm.at[pl.ds(seg, OCAP)], vl_c)
        ng = (cp + LANES - 1) // LANES

        def grp_body(gg, _):
            off = gg * LANES
            valid = (off + iota) < cp
            pk16 = pk_c[pl.ds(off, LANES)]
            v16 = vl_c[pl.ds(off, LANES)]
            locv = jnp.where(valid, lax.shift_right_arithmetic(pk16, 14), 0)
            d16 = pk16 & (PKSH - 1)
            occ, lastm = plsc.scan_count(locv, mask=valid)
            occ0 = occ - occ_base
            old = plsc.load_gather(cnt, [locv], mask=valid)
            pos = old + occ0
            ok = valid & (pos < CAP)
            gidx = jnp.where(ok, locv * CAP + pos, 0)
            plsc.store_scatter(gdst, [gidx], d16, mask=ok)
            plsc.store_scatter(gval, [gidx], v16, mask=ok)
            plsc.addupdate_scatter(cnt, [locv], occ0 + 1, mask=lastm & valid)
            return 0

        lax.fori_loop(0, ng, grp_body, 0)
        return 0

    lax.fori_loop(0, NW, prod_body, 0)

    rbufs = ((rowbuf0, semr0), (rowbuf1, semr1))

    def process_row(r, rowbuf):
        m = jnp.minimum(_sload(cnt, r), CAP)
        o = r * CAP
        nblk = (m + LANES - 1) // LANES

        def rsum(jb, acc):
            vv = (jb * LANES + iota) < m
            return acc + jnp.where(vv, gval[pl.ds(o + jb * LANES, LANES)], 0.0)
        rs = jnp.sum(lax.fori_loop(0, nblk, rsum, zeros16))
        kk = w3s * rs + b3s
        kk = jnp.where(kk >= 0, kk, 0.01 * kk)

        def b1_blk(jb, _):
            jidx = jb * LANES + iota
            jv = jidx < m
            dstj = gdst[pl.ds(o + jb * LANES, LANES)]

            def b1(i2, carry):
                sv16, first16 = carry
                j2a = 2 * i2
                j2b = j2a + 1
                dpa = _sload(gdst, o + j2a)
                vpa = _sload(gval, o + j2a)
                dpb = _sload(gdst, o + j2b)
                vpb = _sload(gval, o + j2b)
                eqa = jv & (dstj == dpa)
                eqb = jv & (dstj == dpb) & (j2b < m)
                sv16 = (sv16 + jnp.where(eqa, vpa, 0.0)
                             + jnp.where(eqb, vpb, 0.0))
                first16 = jnp.minimum(first16,
                                      jnp.where(eqa, j2a, jnp.int32(1 << 30)))
                first16 = jnp.minimum(first16,
                                      jnp.where(eqb, j2b, jnp.int32(1 << 30)))
                return sv16, first16

            sv16, first16 = lax.fori_loop(
                0, (m + 1) // 2, b1,
                (zeros16, jnp.full((LANES,), 1 << 30, jnp.int32)))
            keep16 = jv & (first16 == jidx)
            svm[pl.ds(jb * LANES, LANES)] = jnp.where(keep16, sv16, -1.0)
            return 0

        lax.fori_loop(0, nblk, b1_blk, 0)

        def b2_blk(jb, _):
            dstj = gdst[pl.ds(o + jb * LANES, LANES)]
            svj = svm[pl.ds(jb * LANES, LANES)]
            alive = svj > 0

            def b2(i2, rank16):
                j2a = 2 * i2
                j2b = j2a + 1
                spa = _sload(svm, j2a)
                dpa = _sload(gdst, o + j2a)
                spb = _sload(svm, j2b)
                dpb = _sload(gdst, o + j2b)
                ca = (spa > svj) | ((spa == svj) & (dpa < dstj))
                cb = ((spb > svj) | ((spb == svj) & (dpb < dstj))) & (j2b < m)
                return (rank16 + jnp.where(ca, 1.0, 0.0)
                               + jnp.where(cb, 1.0, 0.0))

            rank16 = lax.fori_loop(0, (m + 1) // 2, b2, zeros16)
            t16 = rank16 - kk
            e2 = jnp.exp(2.0 * t16)
            th = 1.0 - 2.0 / (e2 + 1.0)
            out16 = svj * (1.5 - 0.5 * th)
            plsc.store_scatter(rowbuf, [dstj], out16, mask=alive)
            return 0

        lax.fori_loop(0, nblk, b2_blk, 0)

    def zero_stale(r, rowbuf):
        m = jnp.minimum(_sload(cnt, r), CAP)
        o = r * CAP
        nblk = (m + LANES - 1) // LANES

        def zb(jb, _):
            jv = (jb * LANES + iota) < m
            dstj = gdst[pl.ds(o + jb * LANES, LANES)]
            plsc.store_scatter(rowbuf, [dstj], zeros16, mask=jv)
            return 0

        lax.fori_loop(0, nblk, zb, 0)

    def wait_row(b):
        rowbuf, sem = rbufs[b]
        pltpu.make_async_copy(rowbuf, out_hbm.at[lo], sem).wait()

    def handle_row(p, r, b):
        rowbuf, sem = rbufs[b]

        @pl.when(p > 0)
        def _():
            wait_row(b)
            zero_stale(r - 2, rowbuf)

        process_row(r, rowbuf)
        pltpu.async_copy(rowbuf, out_hbm.at[lo + r], sem)

    def pair_body(p, _):
        r0 = 2 * p
        handle_row(p, r0, 0)

        @pl.when(r0 + 1 < nrows)
        def _():
            handle_row(p, r0 + 1, 1)
        return 0

    lax.fori_loop(0, (nrows + 1) // 2, pair_body, 0)
    wait_row(0)

    @pl.when(nrows > 1)
    def _():
        wait_row(1)


def _rank_scatter(bpack, bval, bcnt, params):
    mesh = plsc.VectorSubcoreMesh(core_axis_name="c", subcore_axis_name="s",
                                  num_cores=NC, num_subcores=NS)
    f = pl.kernel(
        _rank_body,
        out_type=jax.ShapeDtypeStruct((N, N), jnp.float32),
        mesh=mesh,
        compiler_params=pltpu.CompilerParams(needs_layout_passes=False),
        scratch_types=[
            pltpu.VMEM((OCAP,), jnp.int32),
            pltpu.VMEM((OCAP,), jnp.float32),
            pltpu.VMEM((NW * NW + LANES,), jnp.int32),
            pltpu.VMEM((ROWS_PER * CAP + LANES,), jnp.int32),
            pltpu.VMEM((ROWS_PER * CAP + LANES,), jnp.float32),
            pltpu.VMEM((336,), jnp.int32),
            pltpu.VMEM((CAP + LANES,), jnp.float32),
            pltpu.VMEM((N,), jnp.float32),
            pltpu.VMEM((N,), jnp.float32),
            pltpu.VMEM((LANES,), jnp.float32),
            pltpu.SemaphoreType.DMA,
            pltpu.SemaphoreType.DMA,
        ],
    )
    return f(bpack, bval, bcnt, params)


def kernel(x, edge_index, W1, b1, W2, b2, W3, b3):
    h, g = _encode(x, W1, b1, W2)
    epk = edge_index[0] * PKSH + edge_index[1]
    bpack, bval, bcnt = _scores(g, epk, b2)
    params = jnp.zeros((LANES,), jnp.float32)
    params = params.at[0].set(W3[0, 0]).at[1].set(b3[0])
    out = _rank_scatter(bpack, bval, bcnt, params)
    return out, h


# speedup vs baseline: 1.2491x; 1.0530x over previous
import functools

import jax
import jax.numpy as jnp
from jax import lax
from jax.experimental import pallas as pl
from jax.experimental.pallas import tpu as pltpu
from jax.experimental.pallas import tpu_sc as plsc

N = 10000
IN_DIM = 128
LAT = 256
E = 320000

NC = 2
NS = 16
NW = NC * NS
LANES = 16

EPW = E // NW
CHUNK_A = 80
NCHUNK_A = EPW // CHUNK_A

ROWS_PER = (N + NW - 1) // NW
CAP = 128
OCAP = 512
PKSH = 16384


def _lrelu(a):
    return jnp.where(a >= 0, a, 0.01 * a)


def _sload(ref, idx):
    return ref[pl.ds(idx, LANES)][0]


def _encode_body(x_ref, w1_ref, b1_ref, w2_ref, h_ref, g_ref):
    h = _lrelu(jnp.dot(x_ref[...], w1_ref[...],
                       preferred_element_type=jnp.float32) + b1_ref[...])
    h_ref[...] = h
    g_ref[...] = jnp.dot(h, w2_ref[...], preferred_element_type=jnp.float32)


def _encode(x, W1, b1, W2):
    RB = 1000
    grid = (N // RB,)
    return pl.pallas_call(
        _encode_body,
        grid=grid,
        in_specs=[
            pl.BlockSpec((RB, IN_DIM), lambda i: (i, 0)),
            pl.BlockSpec((IN_DIM, LAT), lambda i: (0, 0)),
            pl.BlockSpec((1, LAT), lambda i: (0, 0)),
            pl.BlockSpec((LAT, LAT), lambda i: (0, 0)),
        ],
        out_specs=[
            pl.BlockSpec((RB, LAT), lambda i: (i, 0)),
            pl.BlockSpec((RB, LAT), lambda i: (i, 0)),
        ],
        out_shape=[
            jax.ShapeDtypeStruct((N, LAT), jnp.float32),
            jax.ShapeDtypeStruct((N, LAT), jnp.float32),
        ],
    )(x, W1, b1.reshape(1, LAT), W2)


def _scores_body(g_hbm, epk_hbm, b2_hbm,
                 bpack_hbm, bval_hbm, bcnt_hbm,
                 pkc0, idx_u0, idx_v0, urows0, vrows0,
                 pkc1, idx_u1, idx_v1, urows1, vrows1,
                 b2_v, pkbuf, vlbuf, bcnt, sem0, sem1, semi0, semi1):
    c = lax.axis_index("c")
    s = lax.axis_index("s")
    wid = s * NC + c
    base = wid * EPW
    pltpu.sync_copy(b2_hbm, b2_v)
    iota = lax.iota(jnp.int32, LANES)

    occ_cal, _ = plsc.scan_count(jnp.zeros((LANES,), jnp.int32))
    occ_base = jnp.min(occ_cal)
    for i in range(NW // LANES):
        bcnt[pl.ds(i * LANES, LANES)] = jnp.zeros((LANES,), jnp.int32)

    b2r = [b2_v[pl.ds(j * LANES, LANES)] for j in range(LAT // LANES)]
    bufs = ((pkc0, idx_u0, idx_v0, urows0, vrows0, sem0, semi0),
            (pkc1, idx_u1, idx_v1, urows1, vrows1, sem1, semi1))

    def issue_idx(ci, b):
        pkc, _, _, _, _, _, smi = bufs[b]
        eb = base + ci * CHUNK_A
        pltpu.async_copy(epk_hbm.at[pl.ds(eb, CHUNK_A)], pkc, smi)

    def issue_gather(b):
        pkc, iu, iv, ur, vr, sm, smi = bufs[b]
        pltpu.make_async_copy(epk_hbm.at[pl.ds(0, CHUNK_A)], pkc, smi).wait()
        for gg in range(CHUNK_A // LANES):
            sl = pl.ds(gg * LANES, LANES)
            pk16 = pkc[sl]
            iu[sl] = lax.shift_right_arithmetic(pk16, 14)
            iv[sl] = pk16 & (PKSH - 1)
        pltpu.async_copy(g_hbm.at[iu], ur, sm)
        pltpu.async_copy(g_hbm.at[iv], vr, sm)

    def wait(b):
        _, iu, iv, ur, vr, sm, _ = bufs[b]
        pltpu.make_async_copy(g_hbm.at[iu], ur, sm).wait()
        pltpu.make_async_copy(g_hbm.at[iv], vr, sm).wait()

    def compute(ci, b):
        _, iu, iv, ur, vr, _, _ = bufs[b]
        for gg in range(CHUNK_A // LANES):
            def pair_body(l, sumv):
                e0 = gg * LANES + 2 * l
                e1 = e0 + 1
                acc0 = jnp.zeros((LANES,), jnp.float32)
                acc1 = jnp.zeros((LANES,), jnp.float32)
                for j in range(LAT // LANES):
                    sl = pl.ds(j * LANES, LANES)
                    t0 = ur[e0, sl] - vr[e0, sl] + b2r[j]
                    t1 = ur[e1, sl] - vr[e1, sl] + b2r[j]
                    acc0 = acc0 + jnp.where(t0 >= 0, t0, 0.01 * t0)
                    acc1 = acc1 + jnp.where(t1 >= 0, t1, 0.01 * t1)
                s0 = jnp.sum(acc0)
                s1 = jnp.sum(acc1)
                return (sumv + jnp.where(iota == 2 * l, s0, 0.0)
                             + jnp.where(iota == 2 * l + 1, s1, 0.0))
            sumv = lax.fori_loop(0, LANES // 2, pair_body,
                                 jnp.zeros((LANES,), jnp.float32))
            sig = 1.0 / (1.0 + jnp.exp(-sumv))
            s16 = iu[pl.ds(gg * LANES, LANES)]
            d16 = iv[pl.ds(gg * LANES, LANES)]
            owner = lax.div(s16, ROWS_PER)
            loc = s16 - owner * ROWS_PER
            pack = loc * PKSH + d16
            occ, lastm = plsc.scan_count(owner)
            occ0 = occ - occ_base
            old = plsc.load_gather(bcnt, [owner])
            pos = old + occ0
            ok = pos < OCAP
            bidx = jnp.where(ok, owner * OCAP + pos, 0)
            plsc.store_scatter(pkbuf, [bidx], pack, mask=ok)
            plsc.store_scatter(vlbuf, [bidx], sig, mask=ok)
            plsc.addupdate_scatter(bcnt, [owner], occ0 + 1, mask=lastm)

    issue_idx(0, 0)
    issue_gather(0)
    issue_idx(1, 1)

    def pair_chunks(p, _):
        c0 = 2 * p
        c1 = c0 + 1

        @pl.when(c1 < NCHUNK_A)
        def _():
            issue_gather(1)

        @pl.when(c0 + 2 < NCHUNK_A)
        def _():
            issue_idx(c0 + 2, 0)
        wait(0)
        compute(c0, 0)

        @pl.when(c1 < NCHUNK_A)
        def _():
            @pl.when(c1 + 2 < NCHUNK_A)
            def _():
                issue_idx(c1 + 2, 1)

            @pl.when(c0 + 2 < NCHUNK_A)
            def _():
                issue_gather(0)
            wait(1)
            compute(c1, 1)
        return 0

    lax.fori_loop(0, (NCHUNK_A + 1) // 2, pair_chunks, 0)

    pltpu.sync_copy(pkbuf, bpack_hbm.at[pl.ds(wid * NW * OCAP, NW * OCAP)])
    pltpu.sync_copy(vlbuf, bval_hbm.at[pl.ds(wid * NW * OCAP, NW * OCAP)])
    pltpu.sync_copy(bcnt.at[pl.ds(0, NW)], bcnt_hbm.at[pl.ds(wid * NW, NW)])


def _scores(g, epk, b2):
    mesh = plsc.VectorSubcoreMesh(core_axis_name="c", subcore_axis_name="s",
                                  num_cores=NC, num_subcores=NS)
    f = pl.kernel(
        _scores_body,
        out_type=[
            jax.ShapeDtypeStruct((NW * NW * OCAP,), jnp.int32),
            jax.ShapeDtypeStruct((NW * NW * OCAP,), jnp.float32),
            jax.ShapeDtypeStruct((NW * NW,), jnp.int32),
        ],
        mesh=mesh,
        compiler_params=pltpu.CompilerParams(needs_layout_passes=False),
        scratch_types=[
            pltpu.VMEM((CHUNK_A,), jnp.int32),
            pltpu.VMEM((CHUNK_A,), jnp.int32),
            pltpu.VMEM((CHUNK_A,), jnp.int32),
            pltpu.VMEM((CHUNK_A, LAT), jnp.float32),
            pltpu.VMEM((CHUNK_A, LAT), jnp.float32),
            pltpu.VMEM((CHUNK_A,), jnp.int32),
            pltpu.VMEM((CHUNK_A,), jnp.int32),
            pltpu.VMEM((CHUNK_A,), jnp.int32),
            pltpu.VMEM((CHUNK_A, LAT), jnp.float32),
            pltpu.VMEM((CHUNK_A, LAT), jnp.float32),
            pltpu.VMEM((LAT,), jnp.float32),
            pltpu.VMEM((NW * OCAP,), jnp.int32),
            pltpu.VMEM((NW * OCAP,), jnp.float32),
            pltpu.VMEM((NW + LANES,), jnp.int32),
            pltpu.SemaphoreType.DMA,
            pltpu.SemaphoreType.DMA,
            pltpu.SemaphoreType.DMA,
            pltpu.SemaphoreType.DMA,
        ],
    )
    return f(g, epk, b2)


def _rank_body(bpack_hbm, bval_hbm, bcnt_hbm, params_hbm, out_hbm,
               pk_c, vl_c, bcv, gdst, gval, cnt, svm,
               rowbuf0, rowbuf1, params_v, semr0, semr1):
    c = lax.axis_index("c")
    s = lax.axis_index("s")
    wid = s * NC + c
    lo = wid * ROWS_PER
    nrows = jnp.minimum(ROWS_PER, N - lo)
    iota = lax.iota(jnp.int32, LANES)
    zeros16 = jnp.zeros((LANES,), jnp.float32)

    pltpu.sync_copy(params_hbm, params_v)
    pv = params_v[...]
    w3s = pv[0]
    b3s = pv[1]

    occ_cal, _ = plsc.scan_count(jnp.zeros((LANES,), jnp.int32))
    occ_base = jnp.min(occ_cal)

    for i in range(320 // LANES):
        cnt[pl.ds(i * LANES, LANES)] = jnp.zeros((LANES,), jnp.int32)
    def zrow(i, _):
        rowbuf0[pl.ds(i * LANES, LANES)] = zeros16
        rowbuf1[pl.ds(i * LANES, LANES)] = zeros16
        return 0
    lax.fori_loop(0, N // LANES, zrow, 0)

    pltpu.sync_copy(bcnt_hbm, bcv.at[pl.ds(0, NW * NW)])

    def prod_body(p, _):
        cp = jnp.minimum(_sload(bcv, p * NW + wid), OCAP)
        seg = p * NW * OCAP + wid * OCAP
        pltpu.sync_copy(bpack_hbm.at[pl.ds(seg, OCAP)], pk_c)
        pltpu.sync_copy(bval_hbm.at[pl.ds(seg, OCAP)], vl_c)
        ng = (cp + LANES - 1) // LANES

        def grp_body(gg, _):
            off = gg * LANES
            valid = (off + iota) < cp
            pk16 = pk_c[pl.ds(off, LANES)]
            v16 = vl_c[pl.ds(off, LANES)]
            locv = jnp.where(valid, lax.shift_right_arithmetic(pk16, 14), 0)
            d16 = pk16 & (PKSH - 1)
            occ, lastm = plsc.scan_count(locv, mask=valid)
            occ0 = occ - occ_base
            old = plsc.load_gather(cnt, [locv], mask=valid)
            pos = old + occ0
            ok = valid & (pos < CAP)
            gidx = jnp.where(ok, locv * CAP + pos, 0)
            plsc.store_scatter(gdst, [gidx], d16, mask=ok)
            plsc.store_scatter(gval, [gidx], v16, mask=ok)
            plsc.addupdate_scatter(cnt, [locv], occ0 + 1, mask=lastm & valid)
            return 0

        lax.fori_loop(0, ng, grp_body, 0)
        return 0

    lax.fori_loop(0, NW, prod_body, 0)

    rbufs = ((rowbuf0, semr0), (rowbuf1, semr1))

    def process_row(r, rowbuf):
        m = jnp.minimum(_sload(cnt, r), CAP)
        o = r * CAP
        nblk = (m + LANES - 1) // LANES

        def rsum(jb, acc):
            vv = (jb * LANES + iota) < m
            return acc + jnp.where(vv, gval[pl.ds(o + jb * LANES, LANES)], 0.0)
        rs = jnp.sum(lax.fori_loop(0, nblk, rsum, zeros16))
        kk = w3s * rs + b3s
        kk = jnp.where(kk >= 0, kk, 0.01 * kk)

        def b1_blk(jb, _):
            jidx = jb * LANES + iota
            jv = jidx < m
            dstj = gdst[pl.ds(o + jb * LANES, LANES)]

            def b1(i2, carry):
                sv16, first16 = carry
                j2a = 2 * i2
                j2b = j2a + 1
                dpa = _sload(gdst, o + j2a)
                vpa = _sload(gval, o + j2a)
                dpb = _sload(gdst, o + j2b)
                vpb = _sload(gval, o + j2b)
                eqa = jv & (dstj == dpa)
                eqb = jv & (dstj == dpb) & (j2b < m)
                sv16 = (sv16 + jnp.where(eqa, vpa, 0.0)
                             + jnp.where(eqb, vpb, 0.0))
                first16 = jnp.minimum(first16,
                                      jnp.where(eqa, j2a, jnp.int32(1 << 30)))
                first16 = jnp.minimum(first16,
                                      jnp.where(eqb, j2b, jnp.int32(1 << 30)))
                return sv16, first16

            sv16, first16 = lax.fori_loop(
                0, (m + 1) // 2, b1,
                (zeros16, jnp.full((LANES,), 1 << 30, jnp.int32)))
            keep16 = jv & (first16 == jidx)
            svm[pl.ds(jb * LANES, LANES)] = jnp.where(keep16, sv16, -1.0)
            return 0

        lax.fori_loop(0, nblk, b1_blk, 0)

        def b2_blk(jb, _):
            dstj = gdst[pl.ds(o + jb * LANES, LANES)]
            svj = svm[pl.ds(jb * LANES, LANES)]
            alive = svj > 0

            def b2(i2, rank16):
                j2a = 2 * i2
                j2b = j2a + 1
                spa = _sload(svm, j2a)
                dpa = _sload(gdst, o + j2a)
                spb = _sload(svm, j2b)
                dpb = _sload(gdst, o + j2b)
                ca = (spa > svj) | ((spa == svj) & (dpa < dstj))
                cb = ((spb > svj) | ((spb == svj) & (dpb < dstj))) & (j2b < m)
                return (rank16 + jnp.where(ca, 1.0, 0.0)
                               + jnp.where(cb, 1.0, 0.0))

            rank16 = lax.fori_loop(0, (m + 1) // 2, b2, zeros16)
            t16 = rank16 - kk
            e2 = jnp.exp(2.0 * t16)
            th = 1.0 - 2.0 / (e2 + 1.0)
            out16 = svj * (1.5 - 0.5 * th)
            plsc.store_scatter(rowbuf, [dstj], out16, mask=alive)
            return 0

        lax.fori_loop(0, nblk, b2_blk, 0)

    def zero_stale(r, rowbuf):
        m = jnp.minimum(_sload(cnt, r), CAP)
        o = r * CAP
        nblk = (m + LANES - 1) // LANES

        def zb(jb, _):
            jv = (jb * LANES + iota) < m
            dstj = gdst[pl.ds(o + jb * LANES, LANES)]
            plsc.store_scatter(rowbuf, [dstj], zeros16, mask=jv)
            return 0

        lax.fori_loop(0, nblk, zb, 0)

    def wait_row(b):
        rowbuf, sem = rbufs[b]
        pltpu.make_async_copy(rowbuf, out_hbm.at[lo], sem).wait()

    def handle_row(p, r, b):
        rowbuf, sem = rbufs[b]

        @pl.when(p > 0)
        def _():
            wait_row(b)
            zero_stale(r - 2, rowbuf)

        process_row(r, rowbuf)
        pltpu.async_copy(rowbuf, out_hbm.at[lo + r], sem)

    def pair_body(p, _):
        r0 = 2 * p
        handle_row(p, r0, 0)

        @pl.when(r0 + 1 < nrows)
        def _():
            handle_row(p, r0 + 1, 1)
        return 0

    lax.fori_loop(0, (nrows + 1) // 2, pair_body, 0)
    wait_row(0)

    @pl.when(nrows > 1)
    def _():
        wait_row(1)


def _rank_scatter(bpack, bval, bcnt, params):
    mesh = plsc.VectorSubcoreMesh(core_axis_name="c", subcore_axis_name="s",
                                  num_cores=NC, num_subcores=NS)
    f = pl.kernel(
        _rank_body,
        out_type=jax.ShapeDtypeStruct((N, N), jnp.float32),
        mesh=mesh,
        compiler_params=pltpu.CompilerParams(needs_layout_passes=False),
        scratch_types=[
            pltpu.VMEM((OCAP,), jnp.int32),
            pltpu.VMEM((OCAP,), jnp.float32),
            pltpu.VMEM((NW * NW + LANES,), jnp.int32),
            pltpu.VMEM((ROWS_PER * CAP + LANES,), jnp.int32),
            pltpu.VMEM((ROWS_PER * CAP + LANES,), jnp.float32),
            pltpu.VMEM((336,), jnp.int32),
            pltpu.VMEM((CAP + LANES,), jnp.float32),
            pltpu.VMEM((N,), jnp.float32),
            pltpu.VMEM((N,), jnp.float32),
            pltpu.VMEM((LANES,), jnp.float32),
            pltpu.SemaphoreType.DMA,
            pltpu.SemaphoreType.DMA,
        ],
    )
    return f(bpack, bval, bcnt, params)


def kernel(x, edge_index, W1, b1, W2, b2, W3, b3):
    h, g = _encode(x, W1, b1, W2)
    epk = edge_index[0] * PKSH + edge_index[1]
    bpack, bval, bcnt = _scores(g, epk, b2)
    params = jnp.zeros((LANES,), jnp.float32)
    params = params.at[0].set(W3[0, 0]).at[1].set(b3[0])
    out = _rank_scatter(bpack, bval, bcnt, params)
    return out, h


# speedup vs baseline: 1.2974x; 1.0387x over previous
import functools

import jax
import jax.numpy as jnp
from jax import lax
from jax.experimental import pallas as pl
from jax.experimental.pallas import tpu as pltpu
from jax.experimental.pallas import tpu_sc as plsc

N = 10000
IN_DIM = 128
LAT = 256
E = 320000

NC = 2
NS = 16
NW = NC * NS
LANES = 16

EPW = E // NW
CHUNK_A = 80
NCHUNK_A = EPW // CHUNK_A

ROWS_PER = (N + NW - 1) // NW
CAP = 128
OCAP = 512
PKSH = 16384


def _lrelu(a):
    return jnp.where(a >= 0, a, 0.01 * a)


def _sload(ref, idx):
    return ref[pl.ds(idx, LANES)][0]


def _encode_body(x_ref, w1_ref, b1_ref, w2_ref, h_ref, g_ref):
    h = _lrelu(jnp.dot(x_ref[...], w1_ref[...],
                       preferred_element_type=jnp.float32) + b1_ref[...])
    h_ref[...] = h
    g_ref[...] = jnp.dot(h, w2_ref[...], preferred_element_type=jnp.float32)


def _encode(x, W1, b1, W2):
    RB = 1000
    grid = (N // RB,)
    return pl.pallas_call(
        _encode_body,
        grid=grid,
        in_specs=[
            pl.BlockSpec((RB, IN_DIM), lambda i: (i, 0)),
            pl.BlockSpec((IN_DIM, LAT), lambda i: (0, 0)),
            pl.BlockSpec((1, LAT), lambda i: (0, 0)),
            pl.BlockSpec((LAT, LAT), lambda i: (0, 0)),
        ],
        out_specs=[
            pl.BlockSpec((RB, LAT), lambda i: (i, 0)),
            pl.BlockSpec((RB, LAT), lambda i: (i, 0)),
        ],
        out_shape=[
            jax.ShapeDtypeStruct((N, LAT), jnp.float32),
            jax.ShapeDtypeStruct((N, LAT), jnp.float32),
        ],
    )(x, W1, b1.reshape(1, LAT), W2)


def _scores_body(g_hbm, epk_hbm, b2_hbm,
                 bpack_hbm, bval_hbm, bcnt_hbm,
                 pkc0, idx_u0, idx_v0, urows0, vrows0,
                 pkc1, idx_u1, idx_v1, urows1, vrows1,
                 b2_v, pkbuf, vlbuf, bcnt, sem0, sem1, semi0, semi1):
    c = lax.axis_index("c")
    s = lax.axis_index("s")
    wid = s * NC + c
    base = wid * EPW
    pltpu.sync_copy(b2_hbm, b2_v)
    iota = lax.iota(jnp.int32, LANES)

    occ_cal, _ = plsc.scan_count(jnp.zeros((LANES,), jnp.int32))
    occ_base = jnp.min(occ_cal)
    for i in range(NW // LANES):
        bcnt[pl.ds(i * LANES, LANES)] = jnp.zeros((LANES,), jnp.int32)

    b2r = [b2_v[pl.ds(j * LANES, LANES)] for j in range(LAT // LANES)]
    bufs = ((pkc0, idx_u0, idx_v0, urows0, vrows0, sem0, semi0),
            (pkc1, idx_u1, idx_v1, urows1, vrows1, sem1, semi1))

    def issue_idx(ci, b):
        pkc, _, _, _, _, _, smi = bufs[b]
        eb = base + ci * CHUNK_A
        pltpu.async_copy(epk_hbm.at[pl.ds(eb, CHUNK_A)], pkc, smi)

    def issue_gather(b):
        pkc, iu, iv, ur, vr, sm, smi = bufs[b]
        pltpu.make_async_copy(epk_hbm.at[pl.ds(0, CHUNK_A)], pkc, smi).wait()
        for gg in range(CHUNK_A // LANES):
            sl = pl.ds(gg * LANES, LANES)
            pk16 = pkc[sl]
            iu[sl] = lax.shift_right_arithmetic(pk16, 14)
            iv[sl] = pk16 & (PKSH - 1)
        pltpu.async_copy(g_hbm.at[iu], ur, sm)
        pltpu.async_copy(g_hbm.at[iv], vr, sm)

    def wait(b):
        _, iu, iv, ur, vr, sm, _ = bufs[b]
        pltpu.make_async_copy(g_hbm.at[iu], ur, sm).wait()
        pltpu.make_async_copy(g_hbm.at[iv], vr, sm).wait()

    def compute(ci, b):
        _, iu, iv, ur, vr, _, _ = bufs[b]
        for gg in range(CHUNK_A // LANES):
            def pair_body(l, sumv):
                e0 = gg * LANES + 2 * l
                e1 = e0 + 1
                acc0 = jnp.zeros((LANES,), jnp.float32)
                acc1 = jnp.zeros((LANES,), jnp.float32)
                for j in range(LAT // LANES):
                    sl = pl.ds(j * LANES, LANES)
                    t0 = ur[e0, sl] - vr[e0, sl] + b2r[j]
                    t1 = ur[e1, sl] - vr[e1, sl] + b2r[j]
                    acc0 = acc0 + jnp.where(t0 >= 0, t0, 0.01 * t0)
                    acc1 = acc1 + jnp.where(t1 >= 0, t1, 0.01 * t1)
                s0 = jnp.sum(acc0)
                s1 = jnp.sum(acc1)
                return (sumv + jnp.where(iota == 2 * l, s0, 0.0)
                             + jnp.where(iota == 2 * l + 1, s1, 0.0))
            sumv = lax.fori_loop(0, LANES // 2, pair_body,
                                 jnp.zeros((LANES,), jnp.float32))
            sig = 1.0 / (1.0 + jnp.exp(-sumv))
            s16 = iu[pl.ds(gg * LANES, LANES)]
            d16 = iv[pl.ds(gg * LANES, LANES)]
            owner = lax.div(s16, ROWS_PER)
            loc = s16 - owner * ROWS_PER
            pack = loc * PKSH + d16
            occ, lastm = plsc.scan_count(owner)
            occ0 = occ - occ_base
            old = plsc.load_gather(bcnt, [owner])
            pos = old + occ0
            ok = pos < OCAP
            bidx = jnp.where(ok, owner * OCAP + pos, 0)
            plsc.store_scatter(pkbuf, [bidx], pack, mask=ok)
            plsc.store_scatter(vlbuf, [bidx], sig, mask=ok)
            plsc.addupdate_scatter(bcnt, [owner], occ0 + 1, mask=lastm)

    issue_idx(0, 0)
    issue_gather(0)
    issue_idx(1, 1)

    def pair_chunks(p, _):
        c0 = 2 * p
        c1 = c0 + 1

        @pl.when(c1 < NCHUNK_A)
        def _():
            issue_gather(1)

        @pl.when(c0 + 2 < NCHUNK_A)
        def _():
            issue_idx(c0 + 2, 0)
        wait(0)
        compute(c0, 0)

        @pl.when(c1 < NCHUNK_A)
        def _():
            @pl.when(c1 + 2 < NCHUNK_A)
            def _():
                issue_idx(c1 + 2, 1)

            @pl.when(c0 + 2 < NCHUNK_A)
            def _():
                issue_gather(0)
            wait(1)
            compute(c1, 1)
        return 0

    lax.fori_loop(0, (NCHUNK_A + 1) // 2, pair_chunks, 0)

    pltpu.sync_copy(pkbuf, bpack_hbm.at[pl.ds(wid * NW * OCAP, NW * OCAP)])
    pltpu.sync_copy(vlbuf, bval_hbm.at[pl.ds(wid * NW * OCAP, NW * OCAP)])
    pltpu.sync_copy(bcnt.at[pl.ds(0, NW)], bcnt_hbm.at[pl.ds(wid * NW, NW)])


def _scores(g, epk, b2):
    mesh = plsc.VectorSubcoreMesh(core_axis_name="c", subcore_axis_name="s",
                                  num_cores=NC, num_subcores=NS)
    f = pl.kernel(
        _scores_body,
        out_type=[
            jax.ShapeDtypeStruct((NW * NW * OCAP,), jnp.int32),
            jax.ShapeDtypeStruct((NW * NW * OCAP,), jnp.float32),
            jax.ShapeDtypeStruct((NW * NW,), jnp.int32),
        ],
        mesh=mesh,
        compiler_params=pltpu.CompilerParams(needs_layout_passes=False),
        scratch_types=[
            pltpu.VMEM((CHUNK_A,), jnp.int32),
            pltpu.VMEM((CHUNK_A,), jnp.int32),
            pltpu.VMEM((CHUNK_A,), jnp.int32),
            pltpu.VMEM((CHUNK_A, LAT), jnp.float32),
            pltpu.VMEM((CHUNK_A, LAT), jnp.float32),
            pltpu.VMEM((CHUNK_A,), jnp.int32),
            pltpu.VMEM((CHUNK_A,), jnp.int32),
            pltpu.VMEM((CHUNK_A,), jnp.int32),
            pltpu.VMEM((CHUNK_A, LAT), jnp.float32),
            pltpu.VMEM((CHUNK_A, LAT), jnp.float32),
            pltpu.VMEM((LAT,), jnp.float32),
            pltpu.VMEM((NW * OCAP,), jnp.int32),
            pltpu.VMEM((NW * OCAP,), jnp.float32),
            pltpu.VMEM((NW + LANES,), jnp.int32),
            pltpu.SemaphoreType.DMA,
            pltpu.SemaphoreType.DMA,
            pltpu.SemaphoreType.DMA,
            pltpu.SemaphoreType.DMA,
        ],
    )
    return f(g, epk, b2)


def _rank_body(bpack_hbm, bval_hbm, bcnt_hbm, params_hbm, out_hbm,
               pk_c0, vl_c0, pk_c1, vl_c1, bcv, gdst, gval, cnt, svm,
               rowbuf0, rowbuf1, params_v, semr0, semr1, semp0, semp1):
    c = lax.axis_index("c")
    s = lax.axis_index("s")
    wid = s * NC + c
    lo = wid * ROWS_PER
    nrows = jnp.minimum(ROWS_PER, N - lo)
    iota = lax.iota(jnp.int32, LANES)
    zeros16 = jnp.zeros((LANES,), jnp.float32)

    pltpu.sync_copy(params_hbm, params_v)
    pv = params_v[...]
    w3s = pv[0]
    b3s = pv[1]

    occ_cal, _ = plsc.scan_count(jnp.zeros((LANES,), jnp.int32))
    occ_base = jnp.min(occ_cal)

    for i in range(320 // LANES):
        cnt[pl.ds(i * LANES, LANES)] = jnp.zeros((LANES,), jnp.int32)
    def zrow(i, _):
        rowbuf0[pl.ds(i * LANES, LANES)] = zeros16
        rowbuf1[pl.ds(i * LANES, LANES)] = zeros16
        return 0
    lax.fori_loop(0, N // LANES, zrow, 0)

    pltpu.sync_copy(bcnt_hbm, bcv.at[pl.ds(0, NW * NW)])
    pbufs = ((pk_c0, vl_c0, semp0), (pk_c1, vl_c1, semp1))

    def issue_p(p, b):
        pk_c, vl_c, smp = pbufs[b]
        seg = p * NW * OCAP + wid * OCAP
        pltpu.async_copy(bpack_hbm.at[pl.ds(seg, OCAP)], pk_c, smp)
        pltpu.async_copy(bval_hbm.at[pl.ds(seg, OCAP)], vl_c, smp)

    def wait_p(b):
        pk_c, vl_c, smp = pbufs[b]
        pltpu.make_async_copy(bpack_hbm.at[pl.ds(0, OCAP)], pk_c, smp).wait()
        pltpu.make_async_copy(bval_hbm.at[pl.ds(0, OCAP)], vl_c, smp).wait()

    def drain_p(p, b):
        pk_c, vl_c, _ = pbufs[b]
        cp = jnp.minimum(_sload(bcv, p * NW + wid), OCAP)
        ng = (cp + LANES - 1) // LANES

        def grp_body(gg, _):
            off = gg * LANES
            valid = (off + iota) < cp
            pk16 = pk_c[pl.ds(off, LANES)]
            v16 = vl_c[pl.ds(off, LANES)]
            locv = jnp.where(valid, lax.shift_right_arithmetic(pk16, 14), 0)
            d16 = pk16 & (PKSH - 1)
            occ, lastm = plsc.scan_count(locv, mask=valid)
            occ0 = occ - occ_base
            old = plsc.load_gather(cnt, [locv], mask=valid)
            pos = old + occ0
            ok = valid & (pos < CAP)
            gidx = jnp.where(ok, locv * CAP + pos, 0)
            plsc.store_scatter(gdst, [gidx], d16, mask=ok)
            plsc.store_scatter(gval, [gidx], v16, mask=ok)
            plsc.addupdate_scatter(cnt, [locv], occ0 + 1, mask=lastm & valid)
            return 0

        lax.fori_loop(0, ng, grp_body, 0)

    issue_p(0, 0)

    def prod_pair(p2, _):
        pa = 2 * p2
        pb = pa + 1
        issue_p(pb, 1)
        wait_p(0)
        drain_p(pa, 0)

        @pl.when(pb + 1 < NW)
        def _():
            issue_p(pb + 1, 0)
        wait_p(1)
        drain_p(pb, 1)
        return 0

    lax.fori_loop(0, NW // 2, prod_pair, 0)

    rbufs = ((rowbuf0, semr0), (rowbuf1, semr1))

    def process_row(r, rowbuf):
        m = jnp.minimum(_sload(cnt, r), CAP)
        o = r * CAP
        nblk = (m + LANES - 1) // LANES

        def rsum(jb, acc):
            vv = (jb * LANES + iota) < m
            return acc + jnp.where(vv, gval[pl.ds(o + jb * LANES, LANES)], 0.0)
        rs = jnp.sum(lax.fori_loop(0, nblk, rsum, zeros16))
        kk = w3s * rs + b3s
        kk = jnp.where(kk >= 0, kk, 0.01 * kk)

        def b1_blk(jb, _):
            jidx = jb * LANES + iota
            jv = jidx < m
            dstj = gdst[pl.ds(o + jb * LANES, LANES)]

            def b1(i2, carry):
                sv16, first16 = carry
                j2a = 2 * i2
                j2b = j2a + 1
                dpa = _sload(gdst, o + j2a)
                vpa = _sload(gval, o + j2a)
                dpb = _sload(gdst, o + j2b)
                vpb = _sload(gval, o + j2b)
                eqa = jv & (dstj == dpa)
                eqb = jv & (dstj == dpb) & (j2b < m)
                sv16 = (sv16 + jnp.where(eqa, vpa, 0.0)
                             + jnp.where(eqb, vpb, 0.0))
                first16 = jnp.minimum(first16,
                                      jnp.where(eqa, j2a, jnp.int32(1 << 30)))
                first16 = jnp.minimum(first16,
                                      jnp.where(eqb, j2b, jnp.int32(1 << 30)))
                return sv16, first16

            sv16, first16 = lax.fori_loop(
                0, (m + 1) // 2, b1,
                (zeros16, jnp.full((LANES,), 1 << 30, jnp.int32)))
            keep16 = jv & (first16 == jidx)
            svm[pl.ds(jb * LANES, LANES)] = jnp.where(keep16, sv16, -1.0)
            return 0

        lax.fori_loop(0, nblk, b1_blk, 0)

        def b2_blk(jb, _):
            dstj = gdst[pl.ds(o + jb * LANES, LANES)]
            svj = svm[pl.ds(jb * LANES, LANES)]
            alive = svj > 0

            def b2(i2, rank16):
                j2a = 2 * i2
                j2b = j2a + 1
                spa = _sload(svm, j2a)
                dpa = _sload(gdst, o + j2a)
                spb = _sload(svm, j2b)
                dpb = _sload(gdst, o + j2b)
                ca = (spa > svj) | ((spa == svj) & (dpa < dstj))
                cb = ((spb > svj) | ((spb == svj) & (dpb < dstj))) & (j2b < m)
                return (rank16 + jnp.where(ca, 1.0, 0.0)
                               + jnp.where(cb, 1.0, 0.0))

            rank16 = lax.fori_loop(0, (m + 1) // 2, b2, zeros16)
            t16 = rank16 - kk
            e2 = jnp.exp(2.0 * t16)
            th = 1.0 - 2.0 / (e2 + 1.0)
            out16 = svj * (1.5 - 0.5 * th)
            plsc.store_scatter(rowbuf, [dstj], out16, mask=alive)
            return 0

        lax.fori_loop(0, nblk, b2_blk, 0)

    def zero_stale(r, rowbuf):
        m = jnp.minimum(_sload(cnt, r), CAP)
        o = r * CAP
        nblk = (m + LANES - 1) // LANES

        def zb(jb, _):
            jv = (jb * LANES + iota) < m
            dstj = gdst[pl.ds(o + jb * LANES, LANES)]
            plsc.store_scatter(rowbuf, [dstj], zeros16, mask=jv)
            return 0

        lax.fori_loop(0, nblk, zb, 0)

    def wait_row(b):
        rowbuf, sem = rbufs[b]
        pltpu.make_async_copy(rowbuf, out_hbm.at[lo], sem).wait()

    def handle_row(p, r, b):
        rowbuf, sem = rbufs[b]

        @pl.when(p > 0)
        def _():
            wait_row(b)
            zero_stale(r - 2, rowbuf)

        process_row(r, rowbuf)
        pltpu.async_copy(rowbuf, out_hbm.at[lo + r], sem)

    def pair_body(p, _):
        r0 = 2 * p
        handle_row(p, r0, 0)

        @pl.when(r0 + 1 < nrows)
        def _():
            handle_row(p, r0 + 1, 1)
        return 0

    lax.fori_loop(0, (nrows + 1) // 2, pair_body, 0)
    wait_row(0)

    @pl.when(nrows > 1)
    def _():
        wait_row(1)


def _rank_scatter(bpack, bval, bcnt, params):
    mesh = plsc.VectorSubcoreMesh(core_axis_name="c", subcore_axis_name="s",
                                  num_cores=NC, num_subcores=NS)
    f = pl.kernel(
        _rank_body,
        out_type=jax.ShapeDtypeStruct((N, N), jnp.float32),
        mesh=mesh,
        compiler_params=pltpu.CompilerParams(needs_layout_passes=False),
        scratch_types=[
            pltpu.VMEM((OCAP,), jnp.int32),
            pltpu.VMEM((OCAP,), jnp.float32),
            pltpu.VMEM((OCAP,), jnp.int32),
            pltpu.VMEM((OCAP,), jnp.float32),
            pltpu.VMEM((NW * NW + LANES,), jnp.int32),
            pltpu.VMEM((ROWS_PER * CAP + LANES,), jnp.int32),
            pltpu.VMEM((ROWS_PER * CAP + LANES,), jnp.float32),
            pltpu.VMEM((336,), jnp.int32),
            pltpu.VMEM((CAP + LANES,), jnp.float32),
            pltpu.VMEM((N,), jnp.float32),
            pltpu.VMEM((N,), jnp.float32),
            pltpu.VMEM((LANES,), jnp.float32),
            pltpu.SemaphoreType.DMA,
            pltpu.SemaphoreType.DMA,
            pltpu.SemaphoreType.DMA,
            pltpu.SemaphoreType.DMA,
        ],
    )
    return f(bpack, bval, bcnt, params)


def kernel(x, edge_index, W1, b1, W2, b2, W3, b3):
    h, g = _encode(x, W1, b1, W2)
    epk = edge_index[0] * PKSH + edge_index[1]
    bpack, bval, bcnt = _scores(g, epk, b2)
    params = jnp.zeros((LANES,), jnp.float32)
    params = params.at[0].set(W3[0, 0]).at[1].set(b3[0])
    out = _rank_scatter(bpack, bval, bcnt, params)
    return out, h
